# baseline (speedup 1.0000x reference)
import functools

import jax
import jax.numpy as jnp
from jax import lax
from jax.experimental import pallas as pl
from jax.experimental.pallas import tpu as pltpu
from jax.experimental.pallas import tpu_sc as plsc

N = 2048
E = 65536
D_X = 512
D_E = 16
OUT_X = 512
OUT_E = 256

NC = 2
NS = 16
NW = NC * NS
EPW = E // NW
BCH = 128
KCH = EPW // BCH
RPT = N // NS
L = 16


def _segment_sum_sc(idx2, ea_t):
    mesh = plsc.VectorSubcoreMesh(core_axis_name="c", subcore_axis_name="s")

    @functools.partial(
        pl.kernel,
        out_type=jax.ShapeDtypeStruct((NC * D_E, N), jnp.float32),
        mesh=mesh,
        scratch_types=[
            pltpu.VMEM((EPW,), jnp.int32),
            pltpu.VMEM((KCH, BCH), jnp.int32),
            pltpu.VMEM((D_E, EPW), jnp.float32),
            pltpu.VMEM((EPW, D_E), jnp.float32),
            pltpu.VMEM((RPT, D_E), jnp.float32),
            pltpu.VMEM((D_E, RPT), jnp.float32),
            pltpu.VMEM_SHARED((N, D_E), jnp.float32),
            pltpu.SemaphoreType.DMA,
            pltpu.SemaphoreType.DMA((KCH,)),
            pltpu.SemaphoreType.DMA,
        ],
        compiler_params=pltpu.CompilerParams(use_tc_tiling_on_sc=False,
                                             needs_layout_passes=False),
    )
    def seg_kernel(idx_hbm, ea_hbm, out_hbm, idxf_v, idx_v, tbuf_v, rows_v,
                   stripe_v, stripet_v, acc_sh, sem_idx, sem_ea, sem_sc):
        c = lax.axis_index("c")
        s = lax.axis_index("s")
        wid = s * NC + c
        base = wid * EPW

        with jax.named_scope("stage"):
            cp_idx = pltpu.async_copy(idx_hbm.at[0, pl.ds(base, EPW)],
                                      idxf_v, sem_idx)
            cp_ea = [
                pltpu.async_copy(
                    ea_hbm.at[:, pl.ds(base + j * BCH, BCH)],
                    tbuf_v.at[:, pl.ds(j * BCH, BCH)], sem_ea.at[j])
                for j in range(KCH)
            ]

        with jax.named_scope("zero"):
            def zero_row(i, carry):
                stripe_v[i] = jnp.zeros((D_E,), jnp.float32)
                return carry

            lax.fori_loop(0, RPT, zero_row, 0)
            pltpu.sync_copy(stripe_v, acc_sh.at[pl.ds(s * RPT, RPT)])

        with jax.named_scope("repack"):
            cp_idx.wait()

            def repack(i, carry):
                j = i // (BCH // L)
                b = i % (BCH // L)
                idx_v[j, pl.ds(b * L, L)] = idxf_v[pl.ds(i * L, L)]
                return carry

            lax.fori_loop(0, EPW // L, repack, 0, unroll=2)
        plsc.subcore_barrier()
        lanes = lax.iota(jnp.int32, L)
        dsplats = [jnp.full((L,), d, jnp.int32) for d in range(D_E)]
        CPB = BCH // L

        with jax.named_scope("transpose"):
            def tr_chunk(ch, carry):
                @pl.when(ch % CPB == 0)
                def _wait_stage():
                    j = ch // CPB
                    pltpu.make_async_copy(
                        ea_hbm.at[:, pl.ds(base, BCH)],
                        tbuf_v.at[:, pl.ds(0, BCH)], sem_ea.at[j]).wait()

                erow = ch * L + lanes
                for d in range(D_E):
                    vals = tbuf_v[d, pl.ds(ch * L, L)]
                    plsc.store_scatter(rows_v, [erow, dsplats[d]], vals)

                @pl.when(ch % CPB == CPB - 1)
                def _fire():
                    j = ch // CPB
                    pltpu.async_copy(rows_v.at[pl.ds(j * BCH, BCH)],
                                     acc_sh.at[idx_v.at[j]], sem_sc,
                                     add=True)

                return carry

            lax.fori_loop(0, EPW // L, tr_chunk, 0)

        with jax.named_scope("scatter"):
            def drain(j, carry):
                pltpu.make_async_copy(rows_v.at[pl.ds(0, BCH)],
                                      acc_sh.at[idx_v.at[0]], sem_sc).wait()
                return carry

            lax.fori_loop(0, KCH, drain, 0)
        plsc.subcore_barrier()

        with jax.named_scope("writeout"):
            pltpu.sync_copy(acc_sh.at[pl.ds(s * RPT, RPT)], stripe_v)

            def tr_out(ch, carry):
                rrow = ch * L + lanes
                for d in range(D_E):
                    vals = plsc.load_gather(
                        stripe_v, [rrow, jnp.full((L,), d, jnp.int32)])
                    stripet_v[d, pl.ds(ch * L, L)] = vals
                return carry

            lax.fori_loop(0, RPT // L, tr_out, 0)
            pltpu.sync_copy(stripet_v,
                            out_hbm.at[pl.ds(c * D_E, D_E),
                                       pl.ds(s * RPT, RPT)])

    return seg_kernel(idx2, ea_t)


def _x_linear_tc(x, Wx, bx):
    def body(x_ref, wx_ref, bx_ref, o_ref):
        xo = lax.dot_general(x_ref[...], wx_ref[...],
                             (((1,), (1,)), ((), ())),
                             preferred_element_type=jnp.float32)
        o_ref[...] = xo + bx_ref[...][None, :]

    return pl.pallas_call(
        body,
        out_shape=jax.ShapeDtypeStruct((N, OUT_X + OUT_E), jnp.float32),
        grid=(1,),
        in_specs=[
            pl.BlockSpec((N, D_X), lambda i: (0, 0)),
            pl.BlockSpec((OUT_X, D_X), lambda i: (0, 0)),
            pl.BlockSpec((OUT_X,), lambda i: (0,)),
        ],
        out_specs=pl.BlockSpec((N, OUT_X), lambda i: (0, 0)),
    )(x, Wx, bx)


def _e_linear_tc(xo_full, partials_t, We, be):
    def body(xo_ref, p_ref, we_ref, be_ref, o_ref):
        agg_t = p_ref[:D_E, :] + p_ref[D_E:, :]
        eo = lax.dot_general(agg_t, we_ref[...],
                             (((0,), (1,)), ((), ())),
                             preferred_element_type=jnp.float32)
        o_ref[...] = eo + be_ref[...][None, :]

    return pl.pallas_call(
        body,
        out_shape=jax.ShapeDtypeStruct((N, OUT_X + OUT_E), jnp.float32),
        grid=(1,),
        in_specs=[
            pl.BlockSpec(memory_space=pl.ANY),
            pl.BlockSpec((NC * D_E, N), lambda i: (0, 0)),
            pl.BlockSpec((OUT_E, D_E), lambda i: (0, 0)),
            pl.BlockSpec((OUT_E,), lambda i: (0,)),
        ],
        out_specs=pl.BlockSpec((N, OUT_E), lambda i: (0, 2)),
        input_output_aliases={0: 0},
    )(xo_full, partials_t, We, be)


def kernel(x, edge_index, edge_attr, Wx, bx, We, be):
    ea_t = edge_attr.T
    partials_t = _segment_sum_sc(edge_index.astype(jnp.int32), ea_t)
    xo = _x_linear_tc(x, Wx, bx)
    return _e_linear_tc(xo, partials_t, We, be)


# speedup vs baseline: 1.0264x; 1.0264x over previous
import functools

import jax
import jax.numpy as jnp
from jax import lax
from jax.experimental import pallas as pl
from jax.experimental.pallas import tpu as pltpu
from jax.experimental.pallas import tpu_sc as plsc

N = 2048
E = 65536
D_X = 512
D_E = 16
OUT_X = 512
OUT_E = 256

NC = 2
NS = 16
NW = NC * NS
EPW = E // NW
BCH = 128
KCH = EPW // BCH
RPT = N // NS
L = 16


def _segment_sum_sc(idx2, ea_t):
    mesh = plsc.VectorSubcoreMesh(core_axis_name="c", subcore_axis_name="s")

    @functools.partial(
        pl.kernel,
        out_type=jax.ShapeDtypeStruct((NC * D_E, N), jnp.float32),
        mesh=mesh,
        scratch_types=[
            pltpu.VMEM((EPW,), jnp.int32),
            pltpu.VMEM((KCH, BCH), jnp.int32),
            pltpu.VMEM((D_E, EPW), jnp.float32),
            pltpu.VMEM((EPW, D_E), jnp.float32),
            pltpu.VMEM((RPT, D_E), jnp.float32),
            pltpu.VMEM((D_E, RPT), jnp.float32),
            pltpu.VMEM_SHARED((N, D_E), jnp.float32),
            pltpu.SemaphoreType.DMA,
            pltpu.SemaphoreType.DMA,
            pltpu.SemaphoreType.DMA,
        ],
        compiler_params=pltpu.CompilerParams(use_tc_tiling_on_sc=False,
                                             needs_layout_passes=False),
    )
    def seg_kernel(idx_hbm, ea_hbm, out_hbm, idxf_v, idx_v, tbuf_v, rows_v,
                   stripe_v, stripet_v, acc_sh, sem_idx, sem_ea, sem_sc):
        c = lax.axis_index("c")
        s = lax.axis_index("s")
        wid = s * NC + c
        base = wid * EPW

        with jax.named_scope("stage"):
            cp_idx = pltpu.async_copy(idx_hbm.at[0, pl.ds(base, EPW)],
                                      idxf_v, sem_idx)
            cp_ea = pltpu.async_copy(ea_hbm.at[:, pl.ds(base, EPW)],
                                     tbuf_v, sem_ea)

        with jax.named_scope("zero"):
            def zero_row(i, carry):
                stripe_v[i] = jnp.zeros((D_E,), jnp.float32)
                return carry

            lax.fori_loop(0, RPT, zero_row, 0)
            pltpu.sync_copy(stripe_v, acc_sh.at[pl.ds(s * RPT, RPT)])

        with jax.named_scope("repack"):
            cp_idx.wait()

            def repack(i, carry):
                j = i // (BCH // L)
                b = i % (BCH // L)
                idx_v[j, pl.ds(b * L, L)] = idxf_v[pl.ds(i * L, L)]
                return carry

            lax.fori_loop(0, EPW // L, repack, 0, unroll=2)
        plsc.subcore_barrier()
        lanes = lax.iota(jnp.int32, L)
        dsplats = [jnp.full((L,), d, jnp.int32) for d in range(D_E)]
        CPB = BCH // L

        with jax.named_scope("transpose"):
            cp_ea.wait()

            def tr_chunk(ch, carry):
                erow = ch * L + lanes
                for d in range(D_E):
                    vals = tbuf_v[d, pl.ds(ch * L, L)]
                    plsc.store_scatter(rows_v, [erow, dsplats[d]], vals)

                @pl.when(ch % CPB == CPB - 1)
                def _fire():
                    j = ch // CPB
                    pltpu.async_copy(rows_v.at[pl.ds(j * BCH, BCH)],
                                     acc_sh.at[idx_v.at[j]], sem_sc,
                                     add=True)

                return carry

            lax.fori_loop(0, EPW // L, tr_chunk, 0)

        with jax.named_scope("scatter"):
            def drain(j, carry):
                pltpu.make_async_copy(rows_v.at[pl.ds(0, BCH)],
                                      acc_sh.at[idx_v.at[0]], sem_sc).wait()
                return carry

            lax.fori_loop(0, KCH, drain, 0)
        plsc.subcore_barrier()

        with jax.named_scope("writeout"):
            pltpu.sync_copy(acc_sh.at[pl.ds(s * RPT, RPT)], stripe_v)

            def tr_out(ch, carry):
                rrow = ch * L + lanes
                for d in range(D_E):
                    vals = plsc.load_gather(
                        stripe_v, [rrow, jnp.full((L,), d, jnp.int32)])
                    stripet_v[d, pl.ds(ch * L, L)] = vals
                return carry

            lax.fori_loop(0, RPT // L, tr_out, 0)
            pltpu.sync_copy(stripet_v,
                            out_hbm.at[pl.ds(c * D_E, D_E),
                                       pl.ds(s * RPT, RPT)])

    return seg_kernel(idx2, ea_t)


def _x_linear_tc(x, Wx, bx):
    def body(x_ref, wx_ref, bx_ref, o_ref):
        xo = lax.dot_general(x_ref[...], wx_ref[...],
                             (((1,), (1,)), ((), ())),
                             preferred_element_type=jnp.float32)
        o_ref[...] = xo + bx_ref[...][None, :]

    return pl.pallas_call(
        body,
        out_shape=jax.ShapeDtypeStruct((N, OUT_X + OUT_E), jnp.float32),
        grid=(1,),
        in_specs=[
            pl.BlockSpec((N, D_X), lambda i: (0, 0)),
            pl.BlockSpec((OUT_X, D_X), lambda i: (0, 0)),
            pl.BlockSpec((OUT_X,), lambda i: (0,)),
        ],
        out_specs=pl.BlockSpec((N, OUT_X), lambda i: (0, 0)),
    )(x, Wx, bx)


def _e_linear_tc(xo_full, partials_t, We, be):
    def body(xo_ref, p_ref, we_ref, be_ref, o_ref):
        agg_t = p_ref[:D_E, :] + p_ref[D_E:, :]
        eo = lax.dot_general(agg_t, we_ref[...],
                             (((0,), (1,)), ((), ())),
                             preferred_element_type=jnp.float32)
        o_ref[...] = eo + be_ref[...][None, :]

    return pl.pallas_call(
        body,
        out_shape=jax.ShapeDtypeStruct((N, OUT_X + OUT_E), jnp.float32),
        grid=(1,),
        in_specs=[
            pl.BlockSpec(memory_space=pl.ANY),
            pl.BlockSpec((NC * D_E, N), lambda i: (0, 0)),
            pl.BlockSpec((OUT_E, D_E), lambda i: (0, 0)),
            pl.BlockSpec((OUT_E,), lambda i: (0,)),
        ],
        out_specs=pl.BlockSpec((N, OUT_E), lambda i: (0, 2)),
        input_output_aliases={0: 0},
    )(xo_full, partials_t, We, be)


def kernel(x, edge_index, edge_attr, Wx, bx, We, be):
    ea_t = edge_attr.T
    partials_t = _segment_sum_sc(edge_index.astype(jnp.int32), ea_t)
    xo = _x_linear_tc(x, Wx, bx)
    return _e_linear_tc(xo, partials_t, We, be)


# speedup vs baseline: 1.0277x; 1.0012x over previous
import functools

import jax
import jax.numpy as jnp
from jax import lax
from jax.experimental import pallas as pl
from jax.experimental.pallas import tpu as pltpu
from jax.experimental.pallas import tpu_sc as plsc

N = 2048
E = 65536
D_X = 512
D_E = 16
OUT_X = 512
OUT_E = 256

NC = 2
NS = 16
NW = NC * NS
EPW = E // NW
BCH = 128
KCH = EPW // BCH
RPT = N // NS
L = 16


def _segment_sum_sc(idx2, ea_t):
    mesh = plsc.VectorSubcoreMesh(core_axis_name="c", subcore_axis_name="s")

    @functools.partial(
        pl.kernel,
        out_type=jax.ShapeDtypeStruct((NC * D_E, N), jnp.float32),
        mesh=mesh,
        scratch_types=[
            pltpu.VMEM((EPW,), jnp.int32),
            pltpu.VMEM((KCH, BCH), jnp.int32),
            pltpu.VMEM((D_E, EPW), jnp.float32),
            pltpu.VMEM((EPW, D_E), jnp.float32),
            pltpu.VMEM((RPT, D_E), jnp.float32),
            pltpu.VMEM((D_E, RPT), jnp.float32),
            pltpu.VMEM_SHARED((N, D_E), jnp.float32),
            pltpu.SemaphoreType.DMA,
            pltpu.SemaphoreType.DMA,
            pltpu.SemaphoreType.DMA,
        ],
        compiler_params=pltpu.CompilerParams(use_tc_tiling_on_sc=False,
                                             needs_layout_passes=False,
                                             skip_device_barrier=True),
    )
    def seg_kernel(idx_hbm, ea_hbm, out_hbm, idxf_v, idx_v, tbuf_v, rows_v,
                   stripe_v, stripet_v, acc_sh, sem_idx, sem_ea, sem_sc):
        c = lax.axis_index("c")
        s = lax.axis_index("s")
        wid = s * NC + c
        base = wid * EPW

        with jax.named_scope("stage"):
            cp_idx = pltpu.async_copy(idx_hbm.at[0, pl.ds(base, EPW)],
                                      idxf_v, sem_idx)
            cp_ea = pltpu.async_copy(ea_hbm.at[:, pl.ds(base, EPW)],
                                     tbuf_v, sem_ea)

        with jax.named_scope("zero"):
            def zero_row(i, carry):
                stripe_v[i] = jnp.zeros((D_E,), jnp.float32)
                return carry

            lax.fori_loop(0, RPT, zero_row, 0)
            pltpu.sync_copy(stripe_v, acc_sh.at[pl.ds(s * RPT, RPT)])

        with jax.named_scope("repack"):
            cp_idx.wait()

            def repack(i, carry):
                j = i // (BCH // L)
                b = i % (BCH // L)
                idx_v[j, pl.ds(b * L, L)] = idxf_v[pl.ds(i * L, L)]
                return carry

            lax.fori_loop(0, EPW // L, repack, 0, unroll=2)
        plsc.subcore_barrier()
        lanes = lax.iota(jnp.int32, L)
        dsplats = [jnp.full((L,), d, jnp.int32) for d in range(D_E)]
        CPB = BCH // L

        with jax.named_scope("transpose"):
            cp_ea.wait()

            def tr_chunk(ch, carry):
                erow = ch * L + lanes
                for d in range(D_E):
                    vals = tbuf_v[d, pl.ds(ch * L, L)]
                    plsc.store_scatter(rows_v, [erow, dsplats[d]], vals)

                @pl.when(ch % CPB == CPB - 1)
                def _fire():
                    j = ch // CPB
                    pltpu.async_copy(rows_v.at[pl.ds(j * BCH, BCH)],
                                     acc_sh.at[idx_v.at[j]], sem_sc,
                                     add=True)

                return carry

            lax.fori_loop(0, EPW // L, tr_chunk, 0)

        with jax.named_scope("scatter"):
            def drain(j, carry):
                pltpu.make_async_copy(rows_v.at[pl.ds(0, BCH)],
                                      acc_sh.at[idx_v.at[0]], sem_sc).wait()
                return carry

            lax.fori_loop(0, KCH, drain, 0)
        plsc.subcore_barrier()

        with jax.named_scope("writeout"):
            pltpu.sync_copy(acc_sh.at[pl.ds(s * RPT, RPT)], stripe_v)

            def tr_out(ch, carry):
                rrow = ch * L + lanes
                for d in range(D_E):
                    vals = plsc.load_gather(
                        stripe_v, [rrow, jnp.full((L,), d, jnp.int32)])
                    stripet_v[d, pl.ds(ch * L, L)] = vals
                return carry

            lax.fori_loop(0, RPT // L, tr_out, 0)
            pltpu.sync_copy(stripet_v,
                            out_hbm.at[pl.ds(c * D_E, D_E),
                                       pl.ds(s * RPT, RPT)])

    return seg_kernel(idx2, ea_t)


def _x_linear_tc(x, Wx, bx):
    def body(x_ref, wx_ref, bx_ref, o_ref):
        xo = lax.dot_general(x_ref[...], wx_ref[...],
                             (((1,), (1,)), ((), ())),
                             preferred_element_type=jnp.float32)
        o_ref[...] = xo + bx_ref[...][None, :]

    return pl.pallas_call(
        body,
        out_shape=jax.ShapeDtypeStruct((N, OUT_X + OUT_E), jnp.float32),
        grid=(1,),
        in_specs=[
            pl.BlockSpec((N, D_X), lambda i: (0, 0)),
            pl.BlockSpec((OUT_X, D_X), lambda i: (0, 0)),
            pl.BlockSpec((OUT_X,), lambda i: (0,)),
        ],
        out_specs=pl.BlockSpec((N, OUT_X), lambda i: (0, 0)),
    )(x, Wx, bx)


def _e_linear_tc(xo_full, partials_t, We, be):
    def body(xo_ref, p_ref, we_ref, be_ref, o_ref):
        agg_t = p_ref[:D_E, :] + p_ref[D_E:, :]
        eo = lax.dot_general(agg_t, we_ref[...],
                             (((0,), (1,)), ((), ())),
                             preferred_element_type=jnp.float32)
        o_ref[...] = eo + be_ref[...][None, :]

    return pl.pallas_call(
        body,
        out_shape=jax.ShapeDtypeStruct((N, OUT_X + OUT_E), jnp.float32),
        grid=(1,),
        in_specs=[
            pl.BlockSpec(memory_space=pl.ANY),
            pl.BlockSpec((NC * D_E, N), lambda i: (0, 0)),
            pl.BlockSpec((OUT_E, D_E), lambda i: (0, 0)),
            pl.BlockSpec((OUT_E,), lambda i: (0,)),
        ],
        out_specs=pl.BlockSpec((N, OUT_E), lambda i: (0, 2)),
        input_output_aliases={0: 0},
    )(xo_full, partials_t, We, be)


def kernel(x, edge_index, edge_attr, Wx, bx, We, be):
    ea_t = edge_attr.T
    partials_t = _segment_sum_sc(edge_index.astype(jnp.int32), ea_t)
    xo = _x_linear_tc(x, Wx, bx)
    return _e_linear_tc(xo, partials_t, We, be)
